# Initial kernel scaffold; baseline (speedup 1.0000x reference)
import jax
import jax.numpy as jnp
from jax.experimental import pallas as pl


def kernel(x, edge_index, W1, b1, att1, bias1, bn_gamma, bn_beta, bn_mean, bn_var, W2, b2, att2, bias2):
    raise NotImplementedError("write your pallas kernel here")


# speedup vs baseline: 1.1761x; 1.1761x over previous
import functools

import jax
import jax.numpy as jnp
from jax.experimental import pallas as pl
from jax.experimental.pallas import tpu as pltpu

N = 10000
E = 320000
IN_CH = 128
OUT_CH = 128
HEADS = 8
CPH = 32
H1 = HEADS * CPH


def _mm_bias_body(x_ref, w_ref, b_ref, o_ref):
    o_ref[...] = jnp.dot(x_ref[...], w_ref[...],
                         preferred_element_type=jnp.float32) + b_ref[...]


def _mm_bias(x, w, b):
    n, k = x.shape
    m = w.shape[1]
    return pl.pallas_call(
        _mm_bias_body,
        out_shape=jax.ShapeDtypeStruct((n, m), jnp.float32),
    )(x, w, b[None, :])


def _bn_elu_body(h_ref, sc_ref, of_ref, o_ref):
    v = h_ref[...] * sc_ref[...] + of_ref[...]
    o_ref[...] = jnp.where(v > 0, v, jnp.exp(jnp.minimum(v, 0.0)) - 1.0)


def _bn_elu(h, scale, offset):
    return pl.pallas_call(
        _bn_elu_body,
        out_shape=jax.ShapeDtypeStruct(h.shape, jnp.float32),
    )(h, scale[None, :], offset[None, :])


def _gat_layer_xla(h, src, dst, att, heads, cph):
    n = h.shape[0]
    hh = h.reshape(n, heads, cph)
    e = jax.nn.leaky_relu(hh[src] + hh[dst], negative_slope=0.2)
    alpha = jnp.sum(e * att[None, :, :], axis=-1)
    m = jnp.max(alpha)
    w = jnp.exp(alpha - m)
    denom = jax.ops.segment_sum(w, dst, num_segments=n)
    num = jax.ops.segment_sum(hh[src] * w[:, :, None], dst, num_segments=n)
    return num / (denom[:, :, None] + 1e-38)


def kernel(x, edge_index, W1, b1, att1, bias1, bn_gamma, bn_beta, bn_mean,
           bn_var, W2, b2, att2, bias2):
    n = x.shape[0]
    loop = jnp.arange(n, dtype=edge_index.dtype)
    src = jnp.concatenate([edge_index[0], loop])
    dst = jnp.concatenate([edge_index[1], loop])

    h = _mm_bias(x, W1, b1)
    agg1 = _gat_layer_xla(h, src, dst, att1, HEADS, CPH).reshape(n, H1)

    inv = bn_gamma / jnp.sqrt(bn_var + 1e-5)
    scale = inv
    offset = bias1 * inv + bn_beta - bn_mean * inv
    h2in = _bn_elu(agg1, scale, offset)

    h2 = _mm_bias(h2in, W2, b2)
    agg2 = _gat_layer_xla(h2, src, dst, att2, 1, OUT_CH).reshape(n, OUT_CH)
    return agg2 + bias2


# speedup vs baseline: 5.3826x; 4.5768x over previous
import functools

import jax
import jax.numpy as jnp
from jax import lax
from jax.experimental import pallas as pl
from jax.experimental.pallas import tpu as pltpu
from jax.experimental.pallas import tpu_sc as plsc

N = 10000
E = 320000
IN_CH = 128
OUT_CH = 128
HEADS = 8
CPH = 32
H1 = HEADS * CPH

NC = 2
NS = 16
LN = 16

N_PAD = 10240
E_PAD = 331776
BEA = 128
BEB = 64
ROWS_PER_SUB = N_PAD // NS

_mesh = functools.partial(plsc.VectorSubcoreMesh,
                          core_axis_name="c", subcore_axis_name="s",
                          num_cores=NC, num_subcores=NS)
_params = pltpu.CompilerParams(needs_layout_passes=False)


def _iota16():
    return lax.iota(jnp.int32, LN)


def _full16(v):
    return jnp.full((LN,), v, jnp.int32)


def _make_alpha_kernel(hpc, edge_split, core_off, att_shape):
    nchunks = NC * NS if edge_split else NS
    chunk = E_PAD // nchunks
    nbatch = chunk // BEA
    cpg = 128 // hpc
    blkf = BEA * hpc
    asz = E_PAD * hpc * (1 if edge_split else NC)

    def body(h_hbm, packed_hbm, att_hbm, alpha_hbm, maxes_hbm,
             ib0, ib1, sr0, sr1, dr0, dr1, abuf, attv, mrow,
             ss0, ss1, sd0, sd1):
        c = lax.axis_index("c")
        s = lax.axis_index("s")
        wid = c * NS + s
        w_chunk = wid if edge_split else s
        blk0 = w_chunk * nbatch
        cbase = 0 if edge_split else c * (E_PAD * hpc)
        ibs = (ib0, ib1)
        srs = (sr0, sr1)
        drs = (dr0, dr1)
        sss = (ss0, ss1)
        sds = (sd0, sd1)
        if att_shape[0] == 1:
            pltpu.sync_copy(att_hbm, attv)
        else:
            pltpu.sync_copy(att_hbm.at[pl.ds(c * hpc, hpc)], attv)
        for hh in range(hpc):
            mrow[hh, :] = jnp.full((LN,), -jnp.inf, jnp.float32)

        def issue(b, p):
            pltpu.sync_copy(packed_hbm.at[blk0 + b], ibs[p])
            if core_off:
                off = c * N_PAD

                def adj(i, cr):
                    for r in range(2):
                        ibs[p][r, pl.ds(i * LN, LN)] = (
                            ibs[p][r, pl.ds(i * LN, LN)] + off)
                    return cr

                lax.fori_loop(0, BEA // LN, adj, 0)
            pltpu.async_copy(h_hbm.at[ibs[p].at[0]], srs[p], sss[p])
            pltpu.async_copy(h_hbm.at[ibs[p].at[1]], drs[p], sds[p])

        def compute(b, p):
            pltpu.make_async_copy(
                h_hbm.at[ibs[p].at[0]], srs[p], sss[p]).wait()
            pltpu.make_async_copy(
                h_hbm.at[ibs[p].at[1]], drs[p], sds[p]).wait()

            def group(g, cr):
                el = _iota16() + g * LN
                accs = [jnp.zeros((LN,), jnp.float32) for _ in range(hpc)]
                for blk in range(128 // LN):
                    att_vec = attv[(blk * LN) // cpg,
                                   pl.ds((blk * LN) % cpg, LN)]
                    for k in range(LN):
                        ch = blk * LN + k
                        chv = _full16(ch)
                        av = plsc.load_gather(srs[p], [el, chv])
                        bv = plsc.load_gather(drs[p], [el, chv])
                        t = av + bv
                        t = jnp.maximum(t, 0.2 * t)
                        accs[ch // cpg] = accs[ch // cpg] + t * att_vec[k]
                base = (g // 4) * (hpc * 64) + (g % 4) * LN
                for hh in range(hpc):
                    abuf[pl.ds(base + hh * 64, LN)] = accs[hh]
                    mrow[hh, :] = jnp.maximum(mrow[hh, :], accs[hh])
                return cr

            lax.fori_loop(0, BEA // LN, group, 0)
            pltpu.sync_copy(
                abuf, alpha_hbm.at[pl.ds(cbase + (blk0 + b) * blkf, blkf)])

        issue(0, 0)

        def pair(i, cr):
            for p in range(2):
                b = 2 * i + p

                @pl.when(b + 1 < nbatch)
                def _():
                    issue(b + 1, 1 - p)

                compute(b, p)
            return cr

        lax.fori_loop(0, nbatch // 2, pair, 0)
        if nbatch % 2:
            compute(nbatch - 1, 0)

        m = mrow[0, :]
        for hh in range(1, hpc):
            m = jnp.maximum(m, mrow[hh, :])
        mrow[0, :] = m
        pltpu.sync_copy(mrow.at[0], maxes_hbm.at[wid])

    out_type = (
        jax.ShapeDtypeStruct((asz,), jnp.float32),
        jax.ShapeDtypeStruct((NC * NS, LN), jnp.float32),
    )
    scratch = [
        pltpu.VMEM((2, BEA), jnp.int32),
        pltpu.VMEM((2, BEA), jnp.int32),
        pltpu.VMEM((BEA, 128), jnp.float32),
        pltpu.VMEM((BEA, 128), jnp.float32),
        pltpu.VMEM((BEA, 128), jnp.float32),
        pltpu.VMEM((BEA, 128), jnp.float32),
        pltpu.VMEM((BEA * hpc,), jnp.float32),
        pltpu.VMEM((hpc, cpg), jnp.float32),
        pltpu.VMEM((hpc, LN), jnp.float32),
        pltpu.SemaphoreType.DMA,
        pltpu.SemaphoreType.DMA,
        pltpu.SemaphoreType.DMA,
        pltpu.SemaphoreType.DMA,
    ]
    return pl.kernel(body, out_type=out_type, mesh=_mesh(),
                     compiler_params=_params, scratch_types=scratch)


def _make_agg_kernel(hpc, edge_split, core_off):
    nchunks = NC * NS if edge_split else NS
    chunk = E_PAD // nchunks
    nbatch = chunk // BEB
    cpg = 128 // hpc
    blkf = BEB * hpc
    DROWS = N_PAD // 8
    DRPS = DROWS // NS
    ZR = 16

    def body(h_hbm, packed_hbm, alpha_hbm, maxes_hbm, num_hbm, den_hbm,
             acc_sp, den_sp, ib0, ib1, d80, d81, r0, r1, wb0, wb1,
             abuf, mbuf, sg0, sg1, sw0, sw1):
        c = lax.axis_index("c")
        s = lax.axis_index("s")
        wid = c * NS + s
        w_chunk = wid if edge_split else s
        blk0 = w_chunk * nbatch
        cbase = 0 if edge_split else c * (E_PAD * hpc)
        ibs = (ib0, ib1)
        d8s = (d80, d81)
        rvs = (r0, r1)
        wbs = (wb0, wb1)
        sgs = (sg0, sg1)
        sws = (sw0, sw1)

        def zrow(i, cr):
            for j in range(128 // LN):
                wb0[i, pl.ds(j * LN, LN)] = jnp.zeros((LN,), jnp.float32)
            return cr

        lax.fori_loop(0, BEB, zrow, 0)
        zsrc = wb0.at[pl.ds(0, ZR)]

        def zacc(k, cr):
            pltpu.sync_copy(zsrc,
                            acc_sp.at[pl.ds(s * ROWS_PER_SUB + k * ZR, ZR)])
            return cr

        lax.fori_loop(0, ROWS_PER_SUB // ZR, zacc, 0)

        def zden(k, cr):
            pltpu.sync_copy(zsrc, den_sp.at[pl.ds(s * DRPS + k * ZR, ZR)])
            return cr

        lax.fori_loop(0, DRPS // ZR, zden, 0)
        plsc.subcore_barrier()

        pltpu.sync_copy(maxes_hbm, mbuf)
        m = mbuf[0, :]
        for i in range(1, NC * NS):
            m = jnp.maximum(m, mbuf[i, :])
        M = jnp.max(m)

        def drain(p):
            pltpu.make_async_copy(
                wbs[p], den_sp.at[d8s[p].at[0]], sws[p]).wait()
            pltpu.make_async_copy(
                rvs[p], acc_sp.at[ibs[p].at[1]], sws[p]).wait()

        def issue(b, p):
            if not isinstance(b, int):
                @pl.when(b >= 2)
                def _():
                    drain(p)
            pltpu.sync_copy(packed_hbm.at[blk0 + b], ibs[p])

            def adj(i, cr):
                dv = ibs[p][1, pl.ds(i * LN, LN)]
                d8s[p][0, pl.ds(i * LN, LN)] = lax.shift_right_logical(dv, 3)
                if core_off:
                    ibs[p][0, pl.ds(i * LN, LN)] = (
                        ibs[p][0, pl.ds(i * LN, LN)] + c * N_PAD)
                return cr

            lax.fori_loop(0, BEB // LN, adj, 0)
            pltpu.async_copy(h_hbm.at[ibs[p].at[0]], rvs[p], sgs[p])

        def compute(b, p):
            pltpu.make_async_copy(
                h_hbm.at[ibs[p].at[0]], rvs[p], sgs[p]).wait()

            def zwb(i, cr):
                for j in range(128 // LN):
                    wbs[p][i, pl.ds(j * LN, LN)] = (
                        jnp.zeros((LN,), jnp.float32))
                return cr

            lax.fori_loop(0, BEB, zwb, 0)
            pltpu.sync_copy(
                alpha_hbm.at[pl.ds(cbase + (blk0 + b) * blkf, blkf)], abuf)

            def group(g, cr):
                el = _iota16() + g * LN
                dv = ibs[p][1, pl.ds(g * LN, LN)]
                dcol = (dv & 7) * LN
                for hh in range(hpc):
                    w = jnp.exp(abuf[pl.ds(hh * 64 + g * LN, LN)] - M)
                    plsc.store_scatter(wbs[p], [el, dcol + hh], w)
                    for cc in range(cpg):
                        colv = _full16(hh * cpg + cc)
                        v = plsc.load_gather(rvs[p], [el, colv]) * w
                        plsc.store_scatter(rvs[p], [el, colv], v)
                return cr

            lax.fori_loop(0, BEB // LN, group, 0)
            pltpu.async_copy(wbs[p], den_sp.at[d8s[p].at[0]], sws[p],
                             add=True)
            pltpu.async_copy(rvs[p], acc_sp.at[ibs[p].at[1]], sws[p],
                             add=True)

        issue(0, 0)

        def pairloop(i, cr):
            for p in range(2):
                b = 2 * i + p

                @pl.when(b + 1 < nbatch)
                def _():
                    issue(b + 1, 1 - p)

                compute(b, p)
            return cr

        lax.fori_loop(0, nbatch // 2, pairloop, 0)
        drain(0)
        drain(1)
        plsc.subcore_barrier()

        obuf = r0.at[pl.ds(0, ZR)]

        def cout(k, cr):
            r0_ = s * ROWS_PER_SUB + k * ZR
            pltpu.sync_copy(acc_sp.at[pl.ds(r0_, ZR)], obuf)
            pltpu.sync_copy(obuf, num_hbm.at[pl.ds(c * N_PAD + r0_, ZR)])
            return cr

        lax.fori_loop(0, ROWS_PER_SUB // ZR, cout, 0)

        def dout(k, cr):
            r0_ = s * DRPS + k * ZR
            pltpu.sync_copy(den_sp.at[pl.ds(r0_, ZR)], obuf)
            pltpu.sync_copy(obuf, den_hbm.at[pl.ds(c * DROWS + r0_, ZR)])
            return cr

        lax.fori_loop(0, DRPS // ZR, dout, 0)

    out_type = (
        jax.ShapeDtypeStruct((NC * N_PAD, 128), jnp.float32),
        jax.ShapeDtypeStruct((NC * DROWS, 128), jnp.float32),
    )
    scratch = [
        pltpu.VMEM_SHARED((N_PAD, 128), jnp.float32),
        pltpu.VMEM_SHARED((DROWS, 128), jnp.float32),
        pltpu.VMEM((2, BEB), jnp.int32),
        pltpu.VMEM((2, BEB), jnp.int32),
        pltpu.VMEM((1, BEB), jnp.int32),
        pltpu.VMEM((1, BEB), jnp.int32),
        pltpu.VMEM((BEB, 128), jnp.float32),
        pltpu.VMEM((BEB, 128), jnp.float32),
        pltpu.VMEM((BEB, 128), jnp.float32),
        pltpu.VMEM((BEB, 128), jnp.float32),
        pltpu.VMEM((BEB * hpc,), jnp.float32),
        pltpu.VMEM((NC * NS, LN), jnp.float32),
        pltpu.SemaphoreType.DMA,
        pltpu.SemaphoreType.DMA,
        pltpu.SemaphoreType.DMA,
        pltpu.SemaphoreType.DMA,
    ]
    return pl.kernel(body, out_type=out_type, mesh=_mesh(),
                     compiler_params=_params, scratch_types=scratch)


MMBLK = 1024
NBLK1 = N_PAD // MMBLK


def _mm1_body(x_ref, w_ref, b_ref, o_ref):
    i = pl.program_id(0)
    j = pl.program_id(1)
    v = jnp.dot(x_ref[...], w_ref[...],
                preferred_element_type=jnp.float32) + b_ref[...]
    rid = j * MMBLK + lax.broadcasted_iota(jnp.int32, (MMBLK, 128), 0)
    o_ref[...] = jnp.where(rid < N, v, 0.0)


def _mm1(x, W1, b1):
    return pl.pallas_call(
        _mm1_body,
        grid=(2, NBLK1),
        in_specs=[
            pl.BlockSpec((MMBLK, 128), lambda i, j: (j, 0)),
            pl.BlockSpec((128, 128), lambda i, j: (0, i)),
            pl.BlockSpec((1, 128), lambda i, j: (0, i)),
        ],
        out_specs=pl.BlockSpec((MMBLK, 128), lambda i, j: (i * NBLK1 + j, 0)),
        out_shape=jax.ShapeDtypeStruct((NC * N_PAD, 128), jnp.float32),
    )(x, W1, b1.reshape(1, H1))


def _head_expand():
    col = lax.broadcasted_iota(jnp.int32, (LN, 128), 1)
    row = lax.broadcasted_iota(jnp.int32, (LN, 128), 0)
    return jnp.where(col // CPH == row, 1.0, 0.0).astype(jnp.float32)


def _fin1_body(n0_ref, n1_ref, d0_ref, d1_ref, sc_ref, of_ref, w2_ref,
               b2_ref, o_ref):
    j = pl.program_id(0)
    S = _head_expand()
    div0 = jnp.dot(1.0 / (d0_ref[...] + 1e-20), S,
                   preferred_element_type=jnp.float32)
    div1 = jnp.dot(1.0 / (d1_ref[...] + 1e-20), S,
                   preferred_element_type=jnp.float32)
    agg = jnp.concatenate([n0_ref[...] * div0, n1_ref[...] * div1], axis=1)
    t = agg * sc_ref[...] + of_ref[...]
    h2in = jnp.where(t > 0, t, jnp.exp(jnp.minimum(t, 0.0)) - 1.0)
    v = jnp.dot(h2in, w2_ref[...],
                preferred_element_type=jnp.float32) + b2_ref[...]
    rid = j * MMBLK + lax.broadcasted_iota(jnp.int32, (MMBLK, 128), 0)
    o_ref[...] = jnp.where(rid < N, v, 0.0)


def _fin1(num, den, scale, offset, W2, b2):
    return pl.pallas_call(
        _fin1_body,
        grid=(NBLK1,),
        in_specs=[
            pl.BlockSpec((MMBLK, 128), lambda j: (j, 0)),
            pl.BlockSpec((MMBLK, 128), lambda j: (NBLK1 + j, 0)),
            pl.BlockSpec((MMBLK, LN), lambda j: (j, 0)),
            pl.BlockSpec((MMBLK, LN), lambda j: (NBLK1 + j, 0)),
            pl.BlockSpec((1, H1), lambda j: (0, 0)),
            pl.BlockSpec((1, H1), lambda j: (0, 0)),
            pl.BlockSpec((H1, 128), lambda j: (0, 0)),
            pl.BlockSpec((1, 128), lambda j: (0, 0)),
        ],
        out_specs=pl.BlockSpec((MMBLK, 128), lambda j: (j, 0)),
        out_shape=jax.ShapeDtypeStruct((N_PAD, 128), jnp.float32),
    )(num, num, den, den, scale, offset, W2, b2.reshape(1, OUT_CH))


FBLK = 640
NBLK2 = N_PAD // FBLK


def _fin2_body(n0_ref, n1_ref, d0_ref, d1_ref, b_ref, o_ref):
    row = lax.broadcasted_iota(jnp.int32, (LN, 128), 0)
    S = jnp.where(row == 0, 1.0, 0.0).astype(jnp.float32)
    rec = 1.0 / (d0_ref[...] + d1_ref[...] + 1e-20)
    div = jnp.dot(rec, S, preferred_element_type=jnp.float32)
    o_ref[...] = (n0_ref[...] + n1_ref[...]) * div + b_ref[...]


def _fin2(num, den, bias2):
    return pl.pallas_call(
        _fin2_body,
        grid=(NBLK2,),
        in_specs=[
            pl.BlockSpec((FBLK, 128), lambda j: (j, 0)),
            pl.BlockSpec((FBLK, 128), lambda j: (NBLK2 + j, 0)),
            pl.BlockSpec((FBLK, LN), lambda j: (j, 0)),
            pl.BlockSpec((FBLK, LN), lambda j: (NBLK2 + j, 0)),
            pl.BlockSpec((1, 128), lambda j: (0, 0)),
        ],
        out_specs=pl.BlockSpec((FBLK, 128), lambda j: (j, 0)),
        out_shape=jax.ShapeDtypeStruct((N, OUT_CH), jnp.float32),
    )(num, num, den, den, bias2.reshape(1, OUT_CH))


_make_alpha_kernel = functools.cache(_make_alpha_kernel)
_make_agg_kernel = functools.cache(_make_agg_kernel)


def _pack(src, dst, nchunks, bsz):
    s3 = src.reshape(nchunks, -1, bsz)
    d3 = dst.reshape(nchunks, -1, bsz)
    return jnp.stack([s3, d3], axis=2).reshape(-1, 2, bsz)


def kernel(x, edge_index, W1, b1, att1, bias1, bn_gamma, bn_beta, bn_mean,
           bn_var, W2, b2, att2, bias2):
    loop = jnp.arange(N, dtype=jnp.int32)
    padv = jnp.full((E_PAD - E - N,), N, jnp.int32)
    src = jnp.concatenate([edge_index[0].astype(jnp.int32), loop, padv])
    dst = jnp.concatenate([edge_index[1].astype(jnp.int32), loop, padv])
    pA16 = _pack(src, dst, NS, BEA)
    pB16 = _pack(src, dst, NS, BEB)
    pA32 = _pack(src, dst, NC * NS, BEA)
    pB32 = _pack(src, dst, NC * NS, BEB)

    h_cat = _mm1(x, W1, b1)
    alpha1, maxes1 = _make_alpha_kernel(4, False, True, (HEADS, CPH))(
        h_cat, pA16, att1)
    num1, den1p = _make_agg_kernel(4, False, True)(
        h_cat, pB16, alpha1, maxes1)
    den1 = den1p.reshape(NC * N_PAD, LN)

    inv = bn_gamma * jax.lax.rsqrt(bn_var + 1e-5)
    scale = inv.reshape(1, H1)
    offset = ((bias1 - bn_mean) * inv + bn_beta).reshape(1, H1)
    h2 = _fin1(num1, den1, scale, offset, W2, b2)

    alpha2, maxes2 = _make_alpha_kernel(1, True, False, (1, OUT_CH))(
        h2, pA32, att2)
    num2, den2p = _make_agg_kernel(1, True, False)(
        h2, pB32, alpha2, maxes2)
    den2 = den2p.reshape(NC * N_PAD, LN)
    return _fin2(num2, den2, bias2)
